# baseline (speedup 1.0000x reference)
import functools

import jax
import jax.numpy as jnp
from jax import lax
from jax.experimental import pallas as pl
from jax.experimental.pallas import tpu as pltpu
from jax.experimental.pallas import tpu_sc as plsc

N_NODES = 10000
N_EDGES = 320000
IN_DIM = 128
HIDDEN = 256
OUT_DIM = 128
D = 128

NC, NS = 2, 16
NW = NC * NS
CHUNK = 128
NCHUNK = 80
E_PAD = NW * NCHUNK * CHUNK
TCHUNK = E_PAD // CHUNK
N_PAD = 10240
RPT = N_PAD // NS
NDCH = 2 * E_PAD // (NW * CHUNK)

DH = D // 2
CPS = TCHUNK // NS

_MESH = plsc.VectorSubcoreMesh(
    core_axis_name="c", subcore_axis_name="s", num_cores=NC, num_subcores=NS
)


@functools.partial(
    pl.kernel,
    out_type=jax.ShapeDtypeStruct((NC, 2, N_PAD), jnp.float32),
    mesh=_MESH,
    scratch_types=[
        pltpu.VMEM((NCHUNK, 2, CHUNK), jnp.int32),
        pltpu.VMEM((CHUNK,), jnp.float32),
        pltpu.VMEM_SHARED((N_PAD,), jnp.float32),
        pltpu.VMEM_SHARED((N_PAD,), jnp.float32),
        pltpu.SemaphoreType.DMA,
    ],
    compiler_params=pltpu.CompilerParams(use_tc_tiling_on_sc=False),
)
def _sc_degrees(eidx_hbm, zeros_hbm, degs_hbm, idxb, ones_v,
                deg_out_sh, deg_in_sh, ssem):
    c = lax.axis_index("c")
    s = lax.axis_index("s")
    for k in range(CHUNK // 16):
        ones_v[pl.ds(k * 16, 16)] = jnp.full((16,), 1.0, jnp.float32)

    @pl.when(s == 0)
    def _():
        pltpu.sync_copy(zeros_hbm, deg_out_sh)

    @pl.when(s == 1)
    def _():
        pltpu.sync_copy(zeros_hbm, deg_in_sh)

    plsc.subcore_barrier()
    b0 = (s * NC + c) * NCHUNK
    pltpu.sync_copy(eidx_hbm.at[pl.ds(b0, NCHUNK)], idxb)

    def step(i, carry):
        pltpu.async_copy(ones_v, deg_out_sh.at[idxb.at[i, 0]], ssem, add=True)
        pltpu.async_copy(ones_v, deg_in_sh.at[idxb.at[i, 1]], ssem, add=True)

        @pl.when(i >= 4)
        def _():
            pltpu.make_async_copy(ones_v, deg_out_sh.at[idxb.at[0, 0]], ssem).wait()
            pltpu.make_async_copy(ones_v, deg_out_sh.at[idxb.at[0, 0]], ssem).wait()

        return carry

    lax.fori_loop(0, NCHUNK, step, 0)
    for _k in range(8):
        pltpu.make_async_copy(ones_v, deg_out_sh.at[idxb.at[0, 0]], ssem).wait()
    plsc.subcore_barrier()

    @pl.when(s == 0)
    def _():
        pltpu.sync_copy(deg_out_sh, degs_hbm.at[c, 0])

    @pl.when(s == 1)
    def _():
        pltpu.sync_copy(deg_in_sh, degs_hbm.at[c, 1])


@functools.partial(
    pl.kernel,
    out_type=jax.ShapeDtypeStruct((NC, N_PAD, DH), jnp.float32),
    mesh=_MESH,
    scratch_types=[
        pltpu.VMEM((4, 2, CHUNK), jnp.int32),
        pltpu.VMEM((4, CHUNK, DH), jnp.float32),
        pltpu.VMEM_SHARED((N_PAD, DH), jnp.float32),
        pltpu.VMEM_SHARED((N_PAD, DH), jnp.float32),
        pltpu.SemaphoreType.DMA((4,)),
        pltpu.SemaphoreType.DMA((4,)),
        pltpu.SemaphoreType.DMA((4,)),
    ],
    compiler_params=pltpu.CompilerParams(use_tc_tiling_on_sc=False),
)
def _sc_edge_pass(table_hbm, eidx_hbm, zrows_hbm, parts_hbm,
                  idx2, rows, table_sh, accum_sh, gsem, ssem, isem):
    c = lax.axis_index("c")
    s = lax.axis_index("s")
    r0 = s * RPT
    pltpu.sync_copy(table_hbm.at[c, pl.ds(r0, RPT)], table_sh.at[pl.ds(r0, RPT)])
    pltpu.sync_copy(zrows_hbm.at[pl.ds(r0, RPT)], accum_sh.at[pl.ds(r0, RPT)])
    plsc.subcore_barrier()
    b0 = s * CPS

    def gather(i_slot, chunk):
        pltpu.async_copy(table_sh.at[idx2.at[i_slot, 0]], rows.at[i_slot],
                         gsem.at[i_slot])

    def wait_gather(i_slot):
        pltpu.make_async_copy(table_sh.at[idx2.at[i_slot, 0]], rows.at[i_slot],
                              gsem.at[i_slot]).wait()

    def scatter(i_slot):
        pltpu.async_copy(rows.at[i_slot], accum_sh.at[idx2.at[i_slot, 1]],
                         ssem.at[i_slot], add=True)

    def wait_scatter(i_slot):
        pltpu.make_async_copy(rows.at[i_slot], accum_sh.at[idx2.at[i_slot, 1]],
                              ssem.at[i_slot]).wait()

    def load_idx(i_slot, chunk):
        pltpu.async_copy(eidx_hbm.at[chunk], idx2.at[i_slot], isem.at[i_slot])

    def wait_idx(i_slot):
        pltpu.make_async_copy(eidx_hbm.at[0], idx2.at[i_slot],
                              isem.at[i_slot]).wait()

    pltpu.sync_copy(eidx_hbm.at[b0], idx2.at[0])
    load_idx(1, b0 + 1)
    gather(0, b0)

    def step(g, carry):
        i0 = 4 * g
        for b in range(4):
            i = i0 + b
            wait_gather(b)
            scatter(b)
            if b < 2:
                @pl.when(g > 0)
                def _():
                    wait_scatter((b + 2) % 4)
            else:
                wait_scatter((b + 2) % 4)
            load_idx((b + 2) % 4, b0 + i + 2)
            wait_idx((b + 1) % 4)
            gather((b + 1) % 4, b0 + i + 1)
        return carry

    lax.fori_loop(0, CPS // 4, step, 0)
    wait_scatter(2)
    wait_scatter(3)
    wait_gather(0)
    wait_idx(1)
    plsc.subcore_barrier()
    pltpu.sync_copy(accum_sh.at[pl.ds(r0, RPT)], parts_hbm.at[c, pl.ds(r0, RPT)])


_N_ECH = N_EDGES // CHUNK


def _tc_idx_body(src_ref, dst_ref, eidx_ref):
    eidx_ref[:_N_ECH, 0] = src_ref[...]
    eidx_ref[:_N_ECH, 1] = dst_ref[...]
    eidx_ref[_N_ECH:, 0] = jnp.full((TCHUNK + 2 - _N_ECH, CHUNK), N_NODES,
                                    jnp.int32)
    eidx_ref[_N_ECH:, 1] = jnp.full((TCHUNK + 2 - _N_ECH, CHUNK), N_PAD - 1,
                                    jnp.int32)


_tc_idx = pl.pallas_call(
    _tc_idx_body,
    out_shape=jax.ShapeDtypeStruct((TCHUNK + 2, 2, CHUNK), jnp.int32),
)


def _tc_prep_body(degs_ref, feat_ref, dout_ref, din_ref, hs_ref):
    deg_o = degs_ref[0, 0, :N_NODES] + degs_ref[1, 0, :N_NODES]
    deg_i = degs_ref[0, 1, :N_NODES] + degs_ref[1, 1, :N_NODES]
    dout = lax.rsqrt(jnp.maximum(deg_o, 1.0))
    din = lax.rsqrt(jnp.maximum(deg_i, 1.0))
    dout_ref[...] = dout
    din_ref[...] = din
    hs = feat_ref[...] * dout[:, None]
    hs_ref[0, :N_NODES] = hs[:, :DH]
    hs_ref[1, :N_NODES] = hs[:, DH:]


_tc_prep = pl.pallas_call(
    _tc_prep_body,
    out_shape=(
        jax.ShapeDtypeStruct((N_NODES,), jnp.float32),
        jax.ShapeDtypeStruct((N_NODES,), jnp.float32),
        jax.ShapeDtypeStruct((NC, N_PAD, DH), jnp.float32),
    ),
)


def _tc_mid_body(p_ref, din_ref, dout_ref, W1_ref, b1_ref, g1_ref, be1_ref,
                 a1_ref, W2_ref, t2_ref):
    din = din_ref[...][:, None]
    a0 = p_ref[0, :N_NODES] * din
    a1 = p_ref[1, :N_NODES] * din
    z = (jnp.dot(a0, W1_ref[:DH], preferred_element_type=jnp.float32)
         + jnp.dot(a1, W1_ref[DH:], preferred_element_type=jnp.float32)
         + b1_ref[...])
    mean = jnp.mean(z, axis=0)
    zc = z - mean
    var = jnp.mean(zc * zc, axis=0)
    zn = zc * lax.rsqrt(var + 1e-5) * g1_ref[...] + be1_ref[...]
    h1 = jnp.where(zn > 0, zn, zn * a1_ref[...])
    hs2 = h1 * dout_ref[...][:, None]
    t2_ref[0, :N_NODES] = jnp.dot(hs2, W2_ref[:, :DH],
                                  preferred_element_type=jnp.float32)
    t2_ref[1, :N_NODES] = jnp.dot(hs2, W2_ref[:, DH:],
                                  preferred_element_type=jnp.float32)


_tc_mid = pl.pallas_call(
    _tc_mid_body,
    out_shape=jax.ShapeDtypeStruct((NC, N_PAD, DH), jnp.float32),
)


def _tc_out_body(p_ref, din_ref, b2_ref, g2_ref, be2_ref, a2_ref, out_ref):
    zh = jnp.concatenate([p_ref[0, :N_NODES], p_ref[1, :N_NODES]], axis=1)
    z = zh * din_ref[...][:, None] + b2_ref[...]
    mean = jnp.mean(z, axis=0)
    zc = z - mean
    var = jnp.mean(zc * zc, axis=0)
    zn = zc * lax.rsqrt(var + 1e-5) * g2_ref[...] + be2_ref[...]
    out_ref[...] = jnp.where(zn > 0, zn, zn * a2_ref[...])


_tc_out = pl.pallas_call(
    _tc_out_body,
    out_shape=jax.ShapeDtypeStruct((N_NODES, OUT_DIM), jnp.float32),
)


def kernel(feat, edge_index, W1, b1, gamma1, beta1, a1, W2, b2, gamma2, beta2, a2):
    src = edge_index[0].astype(jnp.int32)
    dst = edge_index[1].astype(jnp.int32)

    eidx = _tc_idx(src.reshape(_N_ECH, CHUNK), dst.reshape(_N_ECH, CHUNK))

    zeros1 = jnp.zeros((N_PAD,), jnp.float32)
    zrows = jnp.zeros((N_PAD, DH), jnp.float32)

    degs = _sc_degrees(eidx, zeros1)
    dout_inv, din_inv, hscaled = _tc_prep(degs, feat)
    p1 = _sc_edge_pass(hscaled, eidx, zrows)
    t2 = _tc_mid(p1, din_inv, dout_inv, W1, b1, gamma1, beta1,
                 a1.reshape(1, 1), W2)
    p2 = _sc_edge_pass(t2, eidx, zrows)
    return _tc_out(p2, din_inv, b2, gamma2, beta2, a2.reshape(1, 1))


# speedup vs baseline: 1.0155x; 1.0155x over previous
import functools

import jax
import jax.numpy as jnp
from jax import lax
from jax.experimental import pallas as pl
from jax.experimental.pallas import tpu as pltpu
from jax.experimental.pallas import tpu_sc as plsc

N_NODES = 10000
N_EDGES = 320000
IN_DIM = 128
HIDDEN = 256
OUT_DIM = 128
D = 128

NC, NS = 2, 16
NW = NC * NS
CHUNK = 128
NCHUNK = 80
E_PAD = NW * NCHUNK * CHUNK
TCHUNK = E_PAD // CHUNK
N_PAD = 10240
RPT = N_PAD // NS
NDCH = 2 * E_PAD // (NW * CHUNK)

DH = D // 2
CPS = TCHUNK // NS

_MESH = plsc.VectorSubcoreMesh(
    core_axis_name="c", subcore_axis_name="s", num_cores=NC, num_subcores=NS
)


@functools.partial(
    pl.kernel,
    out_type=jax.ShapeDtypeStruct((NC, 2, N_PAD), jnp.float32),
    mesh=_MESH,
    scratch_types=[
        pltpu.VMEM((NCHUNK, 2, CHUNK), jnp.int32),
        pltpu.VMEM((CHUNK,), jnp.float32),
        pltpu.VMEM_SHARED((N_PAD,), jnp.float32),
        pltpu.VMEM_SHARED((N_PAD,), jnp.float32),
        pltpu.SemaphoreType.DMA,
    ],
)
def _sc_degrees(eidx_hbm, zeros_hbm, degs_hbm, idxb, ones_v,
                deg_out_sh, deg_in_sh, ssem):
    c = lax.axis_index("c")
    s = lax.axis_index("s")
    for k in range(CHUNK // 16):
        ones_v[pl.ds(k * 16, 16)] = jnp.full((16,), 1.0, jnp.float32)

    @pl.when(s == 0)
    def _():
        pltpu.sync_copy(zeros_hbm, deg_out_sh)

    @pl.when(s == 1)
    def _():
        pltpu.sync_copy(zeros_hbm, deg_in_sh)

    plsc.subcore_barrier()
    b0 = (s * NC + c) * NCHUNK
    pltpu.sync_copy(eidx_hbm.at[pl.ds(b0, NCHUNK)], idxb)

    def step(i, carry):
        pltpu.async_copy(ones_v, deg_out_sh.at[idxb.at[i, 0]], ssem, add=True)
        pltpu.async_copy(ones_v, deg_in_sh.at[idxb.at[i, 1]], ssem, add=True)

        @pl.when(i >= 4)
        def _():
            pltpu.make_async_copy(ones_v, deg_out_sh.at[idxb.at[0, 0]], ssem).wait()
            pltpu.make_async_copy(ones_v, deg_out_sh.at[idxb.at[0, 0]], ssem).wait()

        return carry

    lax.fori_loop(0, NCHUNK, step, 0)
    for _k in range(8):
        pltpu.make_async_copy(ones_v, deg_out_sh.at[idxb.at[0, 0]], ssem).wait()
    plsc.subcore_barrier()

    @pl.when(s == 0)
    def _():
        pltpu.sync_copy(deg_out_sh, degs_hbm.at[c, 0])

    @pl.when(s == 1)
    def _():
        pltpu.sync_copy(deg_in_sh, degs_hbm.at[c, 1])


@functools.partial(
    pl.kernel,
    out_type=jax.ShapeDtypeStruct((NC, N_PAD, DH), jnp.float32),
    mesh=_MESH,
    scratch_types=[
        pltpu.VMEM((4, 2, CHUNK), jnp.int32),
        pltpu.VMEM((4, CHUNK, DH), jnp.float32),
        pltpu.VMEM_SHARED((N_PAD, DH), jnp.float32),
        pltpu.VMEM_SHARED((N_PAD, DH), jnp.float32),
        pltpu.SemaphoreType.DMA((4,)),
        pltpu.SemaphoreType.DMA((4,)),
        pltpu.SemaphoreType.DMA((4,)),
    ],
    compiler_params=pltpu.CompilerParams(use_tc_tiling_on_sc=False),
)
def _sc_edge_pass(table_hbm, eidx_hbm, zrows_hbm, parts_hbm,
                  idx2, rows, table_sh, accum_sh, gsem, ssem, isem):
    c = lax.axis_index("c")
    s = lax.axis_index("s")
    r0 = s * RPT
    pltpu.sync_copy(table_hbm.at[c, pl.ds(r0, RPT)], table_sh.at[pl.ds(r0, RPT)])
    pltpu.sync_copy(zrows_hbm.at[pl.ds(r0, RPT)], accum_sh.at[pl.ds(r0, RPT)])
    plsc.subcore_barrier()
    b0 = s * CPS

    def gather(i_slot, chunk):
        pltpu.async_copy(table_sh.at[idx2.at[i_slot, 0]], rows.at[i_slot],
                         gsem.at[i_slot])

    def wait_gather(i_slot):
        pltpu.make_async_copy(table_sh.at[idx2.at[i_slot, 0]], rows.at[i_slot],
                              gsem.at[i_slot]).wait()

    def scatter(i_slot):
        pltpu.async_copy(rows.at[i_slot], accum_sh.at[idx2.at[i_slot, 1]],
                         ssem.at[i_slot], add=True)

    def wait_scatter(i_slot):
        pltpu.make_async_copy(rows.at[i_slot], accum_sh.at[idx2.at[i_slot, 1]],
                              ssem.at[i_slot]).wait()

    def load_idx(i_slot, chunk):
        pltpu.async_copy(eidx_hbm.at[chunk], idx2.at[i_slot], isem.at[i_slot])

    def wait_idx(i_slot):
        pltpu.make_async_copy(eidx_hbm.at[0], idx2.at[i_slot],
                              isem.at[i_slot]).wait()

    pltpu.sync_copy(eidx_hbm.at[b0], idx2.at[0])
    load_idx(1, b0 + 1)
    gather(0, b0)

    def step(g, carry):
        i0 = 4 * g
        for b in range(4):
            i = i0 + b
            wait_gather(b)
            scatter(b)
            if b < 2:
                @pl.when(g > 0)
                def _():
                    wait_scatter((b + 2) % 4)
            else:
                wait_scatter((b + 2) % 4)
            load_idx((b + 2) % 4, b0 + i + 2)
            wait_idx((b + 1) % 4)
            gather((b + 1) % 4, b0 + i + 1)
        return carry

    lax.fori_loop(0, CPS // 4, step, 0)
    wait_scatter(2)
    wait_scatter(3)
    wait_gather(0)
    wait_idx(1)
    plsc.subcore_barrier()
    pltpu.sync_copy(accum_sh.at[pl.ds(r0, RPT)], parts_hbm.at[c, pl.ds(r0, RPT)])


def _tc_prep_body(degs_ref, feat_ref, dout_ref, din_ref, hs_ref):
    deg_o = degs_ref[0, 0, :N_NODES] + degs_ref[1, 0, :N_NODES]
    deg_i = degs_ref[0, 1, :N_NODES] + degs_ref[1, 1, :N_NODES]
    dout = lax.rsqrt(jnp.maximum(deg_o, 1.0))
    din = lax.rsqrt(jnp.maximum(deg_i, 1.0))
    dout_ref[...] = dout
    din_ref[...] = din
    hs = feat_ref[...] * dout[:, None]
    hs_ref[0, :N_NODES] = hs[:, :DH]
    hs_ref[1, :N_NODES] = hs[:, DH:]


_tc_prep = pl.pallas_call(
    _tc_prep_body,
    out_shape=(
        jax.ShapeDtypeStruct((N_NODES,), jnp.float32),
        jax.ShapeDtypeStruct((N_NODES,), jnp.float32),
        jax.ShapeDtypeStruct((NC, N_PAD, DH), jnp.float32),
    ),
)


def _tc_mid_body(p_ref, din_ref, dout_ref, W1_ref, b1_ref, g1_ref, be1_ref,
                 a1_ref, W2_ref, t2_ref):
    din = din_ref[...][:, None]
    a0 = p_ref[0, :N_NODES] * din
    a1 = p_ref[1, :N_NODES] * din
    z = (jnp.dot(a0, W1_ref[:DH], preferred_element_type=jnp.float32)
         + jnp.dot(a1, W1_ref[DH:], preferred_element_type=jnp.float32)
         + b1_ref[...])
    mean = jnp.mean(z, axis=0)
    zc = z - mean
    var = jnp.mean(zc * zc, axis=0)
    zn = zc * lax.rsqrt(var + 1e-5) * g1_ref[...] + be1_ref[...]
    h1 = jnp.where(zn > 0, zn, zn * a1_ref[...])
    hs2 = h1 * dout_ref[...][:, None]
    t2_ref[0, :N_NODES] = jnp.dot(hs2, W2_ref[:, :DH],
                                  preferred_element_type=jnp.float32)
    t2_ref[1, :N_NODES] = jnp.dot(hs2, W2_ref[:, DH:],
                                  preferred_element_type=jnp.float32)


_tc_mid = pl.pallas_call(
    _tc_mid_body,
    out_shape=jax.ShapeDtypeStruct((NC, N_PAD, DH), jnp.float32),
)


def _tc_out_body(p_ref, din_ref, b2_ref, g2_ref, be2_ref, a2_ref, out_ref):
    zh = jnp.concatenate([p_ref[0, :N_NODES], p_ref[1, :N_NODES]], axis=1)
    z = zh * din_ref[...][:, None] + b2_ref[...]
    mean = jnp.mean(z, axis=0)
    zc = z - mean
    var = jnp.mean(zc * zc, axis=0)
    zn = zc * lax.rsqrt(var + 1e-5) * g2_ref[...] + be2_ref[...]
    out_ref[...] = jnp.where(zn > 0, zn, zn * a2_ref[...])


_tc_out = pl.pallas_call(
    _tc_out_body,
    out_shape=jax.ShapeDtypeStruct((N_NODES, OUT_DIM), jnp.float32),
)


def kernel(feat, edge_index, W1, b1, gamma1, beta1, a1, W2, b2, gamma2, beta2, a2):
    src = edge_index[0].astype(jnp.int32)
    dst = edge_index[1].astype(jnp.int32)
    n_pad_e = E_PAD - N_EDGES

    src_p = jnp.concatenate([src, jnp.full((n_pad_e,), N_NODES, jnp.int32)])
    dst_p = jnp.concatenate([dst, jnp.full((n_pad_e,), N_PAD - 1, jnp.int32)])
    eidx = jnp.stack([src_p.reshape(-1, CHUNK), dst_p.reshape(-1, CHUNK)], 1)
    eidx = jnp.concatenate([eidx, jnp.zeros((2, 2, CHUNK), jnp.int32)])

    zeros1 = jnp.zeros((N_PAD,), jnp.float32)
    zrows = jnp.zeros((N_PAD, DH), jnp.float32)

    degs = _sc_degrees(eidx, zeros1)
    dout_inv, din_inv, hscaled = _tc_prep(degs, feat)
    p1 = _sc_edge_pass(hscaled, eidx, zrows)
    t2 = _tc_mid(p1, din_inv, dout_inv, W1, b1, gamma1, beta1,
                 a1.reshape(1, 1), W2)
    p2 = _sc_edge_pass(t2, eidx, zrows)
    return _tc_out(p2, din_inv, b2, gamma2, beta2, a2.reshape(1, 1))
